# baseline (speedup 1.0000x reference)
import functools

import jax
import jax.numpy as jnp
from jax import lax
from jax.experimental import pallas as pl
from jax.experimental.pallas import tpu as pltpu
from jax.experimental.pallas import tpu_sc as plsc

HIDDEN = 768
QUARTER = HIDDEN // 4
BATCH = 16384

ROWS = 96
NC, NS, L = 2, 16, 16
NW = NC * NS
BPW = BATCH // NW
GROUPS = BPW // L
UNROLL = 8


def _proj_body(t_ref, w_ref, b_ref, o_ref):
    acc = lax.dot_general(
        t_ref[...], w_ref[...], (((1,), (1,)), ((), ())),
        preferred_element_type=jnp.float32)
    row = lax.broadcasted_iota(jnp.int32, (ROWS, 1), 0)
    o_ref[...] = acc + jnp.where(row < 24, b_ref[...], 0.0)


def _sc_body(pt_hbm, ih_hbm, id_hbm, iw_hbm, im_hbm, out_hbm,
             pt_v, ih_v, id_v, iw_v, im_v, ob_v, sem):
    wid = lax.axis_index("s") * NC + lax.axis_index("c")
    base = wid * BPW
    pltpu.sync_copy(pt_hbm, pt_v)
    pltpu.sync_copy(ih_hbm.at[pl.ds(base, BPW)], ih_v)
    pltpu.sync_copy(id_hbm.at[pl.ds(base, BPW)], id_v)
    pltpu.sync_copy(iw_hbm.at[pl.ds(base, BPW)], iw_v)
    pltpu.sync_copy(im_hbm.at[pl.ds(base, BPW)], im_v)

    riota = lax.iota(jnp.int32, L)

    def group_body(g, _):
        row0 = g * L
        bh = ih_v[pl.ds(row0, L)]
        bd = id_v[pl.ds(row0, L)]
        bw = iw_v[pl.ds(row0, L)]
        bm = im_v[pl.ds(row0, L)]

        def col_body(j, _):
            for u in range(UNROLL):
                c = j * UNROLL + u
                vh = plsc.load_gather(pt_v, [bh + c])
                vd = plsc.load_gather(pt_v, [bd + c])
                vw = plsc.load_gather(pt_v, [bw + c])
                vm = plsc.load_gather(pt_v, [bm + c])
                cv = riota * 0 + c
                plsc.store_scatter(ob_v, [riota, cv], (vh + vd) + (vw + vm))
            return 0

        lax.fori_loop(0, HIDDEN // UNROLL, col_body, 0)
        pltpu.sync_copy(ob_v, out_hbm.at[pl.ds(base + row0, L)])
        return 0

    lax.fori_loop(0, GROUPS, group_body, 0)


@jax.jit
def kernel(hours, days, weeks, months, hour_table, day_table, week_table,
           month_table, proj_w, proj_b):
    f32 = jnp.float32
    t = jnp.zeros((ROWS, HIDDEN), f32)
    t = t.at[0:24, 0:QUARTER].set(hour_table)
    t = t.at[24:31, QUARTER:2 * QUARTER].set(day_table)
    t = t.at[31:83, 2 * QUARTER:3 * QUARTER].set(week_table)
    t = t.at[83:95, 3 * QUARTER:4 * QUARTER].set(month_table)

    ptable = pl.pallas_call(
        _proj_body,
        out_shape=jax.ShapeDtypeStruct((ROWS, HIDDEN), f32),
    )(t, proj_w, proj_b.reshape(1, HIDDEN))

    i32 = jnp.int32
    ih = hours.astype(i32) * HIDDEN
    idd = (days.astype(i32) + 24) * HIDDEN
    iw = (weeks.astype(i32) + 31) * HIDDEN
    im = (months.astype(i32) + 83) * HIDDEN

    mesh = plsc.VectorSubcoreMesh(core_axis_name="c", subcore_axis_name="s")
    sc = functools.partial(
        pl.kernel,
        out_type=jax.ShapeDtypeStruct((BATCH, HIDDEN), f32),
        mesh=mesh,
        compiler_params=pltpu.CompilerParams(needs_layout_passes=False),
        scratch_types=[
            pltpu.VMEM((ROWS * HIDDEN,), f32),
            pltpu.VMEM((BPW,), i32),
            pltpu.VMEM((BPW,), i32),
            pltpu.VMEM((BPW,), i32),
            pltpu.VMEM((BPW,), i32),
            pltpu.VMEM((L, HIDDEN), f32),
            pltpu.SemaphoreType.DMA,
        ],
    )(_sc_body)
    return sc(ptable.reshape(ROWS * HIDDEN), ih, idd, iw, im)


# speedup vs baseline: 1.6915x; 1.6915x over previous
import functools

import jax
import jax.numpy as jnp
from jax import lax
from jax.experimental import pallas as pl
from jax.experimental.pallas import tpu as pltpu
from jax.experimental.pallas import tpu_sc as plsc

HIDDEN = 768
QUARTER = HIDDEN // 4
BATCH = 16384

ROWS = 96
NC, NS, L = 2, 16, 16
NW = NC * NS
BPW = BATCH // NW
GROUPS = BPW // L
UNROLL = 8


def _proj_body(t_ref, w_ref, b_ref, o_ref):
    acc = lax.dot_general(
        t_ref[...], w_ref[...], (((1,), (1,)), ((), ())),
        preferred_element_type=jnp.float32)
    row = lax.broadcasted_iota(jnp.int32, (ROWS, 1), 0)
    o_ref[...] = acc + jnp.where(row < 24, b_ref[...], 0.0)


def _sc_body(pt_hbm, ih_hbm, id_hbm, iw_hbm, im_hbm, out_hbm,
             pt_v, ih_v, id_v, iw_v, im_v, ob_v, sem):
    wid = lax.axis_index("s") * NC + lax.axis_index("c")
    base = wid * BPW
    pltpu.sync_copy(pt_hbm, pt_v)
    pltpu.sync_copy(ih_hbm.at[pl.ds(base, BPW)], ih_v)
    pltpu.sync_copy(id_hbm.at[pl.ds(base, BPW)], id_v)
    pltpu.sync_copy(iw_hbm.at[pl.ds(base, BPW)], iw_v)
    pltpu.sync_copy(im_hbm.at[pl.ds(base, BPW)], im_v)

    riota = lax.iota(jnp.int32, L)

    def group_body(g, _):
        row0 = g * L
        bh = ih_v[pl.ds(row0, L)]
        bd = id_v[pl.ds(row0, L)]
        bw = iw_v[pl.ds(row0, L)]
        bm = im_v[pl.ds(row0, L)]

        @plsc.parallel_loop(0, HIDDEN, unroll=UNROLL)
        def col_body(c):
            vh = plsc.load_gather(pt_v, [bh + c])
            vd = plsc.load_gather(pt_v, [bd + c])
            vw = plsc.load_gather(pt_v, [bw + c])
            vm = plsc.load_gather(pt_v, [bm + c])
            cv = riota * 0 + c
            plsc.store_scatter(ob_v, [riota, cv], (vh + vd) + (vw + vm))
        pltpu.sync_copy(ob_v, out_hbm.at[pl.ds(base + row0, L)])
        return 0

    lax.fori_loop(0, GROUPS, group_body, 0)


@jax.jit
def kernel(hours, days, weeks, months, hour_table, day_table, week_table,
           month_table, proj_w, proj_b):
    f32 = jnp.float32
    t = jnp.zeros((ROWS, HIDDEN), f32)
    t = t.at[0:24, 0:QUARTER].set(hour_table)
    t = t.at[24:31, QUARTER:2 * QUARTER].set(day_table)
    t = t.at[31:83, 2 * QUARTER:3 * QUARTER].set(week_table)
    t = t.at[83:95, 3 * QUARTER:4 * QUARTER].set(month_table)

    ptable = pl.pallas_call(
        _proj_body,
        out_shape=jax.ShapeDtypeStruct((ROWS, HIDDEN), f32),
    )(t, proj_w, proj_b.reshape(1, HIDDEN))

    i32 = jnp.int32
    ih = hours.astype(i32) * HIDDEN
    idd = (days.astype(i32) + 24) * HIDDEN
    iw = (weeks.astype(i32) + 31) * HIDDEN
    im = (months.astype(i32) + 83) * HIDDEN

    mesh = plsc.VectorSubcoreMesh(core_axis_name="c", subcore_axis_name="s")
    sc = functools.partial(
        pl.kernel,
        out_type=jax.ShapeDtypeStruct((BATCH, HIDDEN), f32),
        mesh=mesh,
        compiler_params=pltpu.CompilerParams(needs_layout_passes=False),
        scratch_types=[
            pltpu.VMEM((ROWS * HIDDEN,), f32),
            pltpu.VMEM((BPW,), i32),
            pltpu.VMEM((BPW,), i32),
            pltpu.VMEM((BPW,), i32),
            pltpu.VMEM((BPW,), i32),
            pltpu.VMEM((L, HIDDEN), f32),
            pltpu.SemaphoreType.DMA,
        ],
    )(_sc_body)
    return sc(ptable.reshape(ROWS * HIDDEN), ih, idd, iw, im)


# speedup vs baseline: 12.3286x; 7.2884x over previous
import functools

import jax
import jax.numpy as jnp
import numpy as np
from jax import lax
from jax.experimental import pallas as pl
from jax.experimental.pallas import tpu as pltpu
from jax.experimental.pallas import tpu_sc as plsc

HIDDEN = 768
QUARTER = HIDDEN // 4
BATCH = 16384

ROWS = 96
NHD = 24 * 7
NWM = 52 * 12
NFULL = NHD + NWM
NC, NS, L = 2, 16, 16
NW = NC * NS
BPW = BATCH // NW
G = 32
CHUNKS = BPW // G
CVECS = HIDDEN // L

_E = np.zeros((NFULL, ROWS), np.float32)
for _i in range(NHD):
    _E[_i, _i // 7] = 1.0
    _E[_i, 24 + _i % 7] = 1.0
for _i in range(NWM):
    _E[NHD + _i, 31 + _i // 12] = 1.0
    _E[NHD + _i, 83 + _i % 12] = 1.0


def _proj_body(t_ref, w_ref, e_ref, b_ref, o_ref):
    pt = lax.dot_general(
        t_ref[...], w_ref[...], (((1,), (1,)), ((), ())),
        preferred_element_type=jnp.float32)
    full = lax.dot_general(
        e_ref[...], pt, (((1,), (0,)), ((), ())),
        preferred_element_type=jnp.float32)
    row = lax.broadcasted_iota(jnp.int32, (NFULL, 1), 0)
    o_ref[...] = full + jnp.where(row < NHD, b_ref[...], 0.0)


def _sc_body(ptf_hbm, ihd_hbm, iwm_hbm, out_hbm,
             ihd_v, iwm_v, a0, a1, b0, b1, sem_g, sem_o):
    wid = lax.axis_index("s") * NC + lax.axis_index("c")
    base = wid * BPW
    pltpu.sync_copy(ihd_hbm.at[pl.ds(base, BPW)], ihd_v)
    pltpu.sync_copy(iwm_hbm.at[pl.ds(base, BPW)], iwm_v)

    abufs = (a0, a1)
    bbufs = (b0, b1)

    def start_gathers(t):
        a, b = abufs[t % 2], bbufs[t % 2]
        pltpu.async_copy(ptf_hbm.at[ihd_v.at[pl.ds(t * G, G)]], a, sem_g)
        pltpu.async_copy(ptf_hbm.at[iwm_v.at[pl.ds(t * G, G)]], b, sem_g)

    def wait_one(sem, dst):
        pltpu.make_async_copy(ptf_hbm.at[pl.ds(0, G)], dst, sem).wait()

    start_gathers(0)
    for t in range(CHUNKS):
        a, b = abufs[t % 2], bbufs[t % 2]
        if t + 1 < CHUNKS:
            if t >= 1:
                wait_one(sem_o, abufs[(t + 1) % 2])
            start_gathers(t + 1)
        wait_one(sem_g, a)
        wait_one(sem_g, b)

        @plsc.parallel_loop(0, G)
        def row_body(r):
            for c in range(CVECS):
                x = b[r, pl.ds(c * L, L)]
                plsc.addupdate(a.at[r, pl.ds(c * L, L)], x)

        pltpu.async_copy(a, out_hbm.at[pl.ds(base + t * G, G)], sem_o)
    wait_one(sem_o, a0)
    wait_one(sem_o, a1)


@jax.jit
def kernel(hours, days, weeks, months, hour_table, day_table, week_table,
           month_table, proj_w, proj_b):
    f32 = jnp.float32
    t = jnp.zeros((ROWS, HIDDEN), f32)
    t = t.at[0:24, 0:QUARTER].set(hour_table)
    t = t.at[24:31, QUARTER:2 * QUARTER].set(day_table)
    t = t.at[31:83, 2 * QUARTER:3 * QUARTER].set(week_table)
    t = t.at[83:95, 3 * QUARTER:4 * QUARTER].set(month_table)

    ptable = pl.pallas_call(
        _proj_body,
        out_shape=jax.ShapeDtypeStruct((NFULL, HIDDEN), f32),
    )(t, proj_w, jnp.asarray(_E), proj_b.reshape(1, HIDDEN))

    i32 = jnp.int32
    ihd = hours.astype(i32) * 7 + days.astype(i32)
    iwm = NHD + weeks.astype(i32) * 12 + months.astype(i32)

    mesh = plsc.VectorSubcoreMesh(core_axis_name="c", subcore_axis_name="s")
    sc = functools.partial(
        pl.kernel,
        out_type=jax.ShapeDtypeStruct((BATCH, HIDDEN), f32),
        mesh=mesh,
        compiler_params=pltpu.CompilerParams(needs_layout_passes=False),
        scratch_types=[
            pltpu.VMEM((BPW,), i32),
            pltpu.VMEM((BPW,), i32),
            pltpu.VMEM((G, HIDDEN), f32),
            pltpu.VMEM((G, HIDDEN), f32),
            pltpu.VMEM((G, HIDDEN), f32),
            pltpu.VMEM((G, HIDDEN), f32),
            pltpu.SemaphoreType.DMA,
            pltpu.SemaphoreType.DMA,
        ],
    )(_sc_body)
    return sc(ptable, ihd, iwm)


# speedup vs baseline: 14.4187x; 1.1695x over previous
import functools

import jax
import jax.numpy as jnp
import numpy as np
from jax import lax
from jax.experimental import pallas as pl
from jax.experimental.pallas import tpu as pltpu
from jax.experimental.pallas import tpu_sc as plsc

HIDDEN = 768
QUARTER = HIDDEN // 4
BATCH = 16384

ROWS = 96
NHD = 24 * 7
NWM = 52 * 12
NFULL = NHD + NWM
NC, NS, L = 2, 16, 16
NW = NC * NS
BPW = BATCH // NW
G = 32
CHUNKS = BPW // G

_E = np.zeros((NFULL, ROWS), np.float32)
for _i in range(NHD):
    _E[_i, _i // 7] = 1.0
    _E[_i, 24 + _i % 7] = 1.0
for _i in range(NWM):
    _E[NHD + _i, 31 + _i // 12] = 1.0
    _E[NHD + _i, 83 + _i % 12] = 1.0


def _proj_body(t_ref, w_ref, e_ref, b_ref, o_ref):
    pt = lax.dot_general(
        t_ref[...], w_ref[...], (((1,), (1,)), ((), ())),
        preferred_element_type=jnp.float32)
    full = lax.dot_general(
        e_ref[...], pt, (((1,), (0,)), ((), ())),
        preferred_element_type=jnp.float32)
    row = lax.broadcasted_iota(jnp.int32, (NFULL, 1), 0)
    o_ref[...] = (full + jnp.where(row < NHD, b_ref[...], 0.0)).astype(
        jnp.bfloat16)


def _sc_body(ptf_hbm, ihd_hbm, iwm_hbm, out_hbm,
             ihd_v, iwm_v, a0, a1, b0, b1, o0, o1, sem_g, sem_o):
    wid = lax.axis_index("s") * NC + lax.axis_index("c")
    base = wid * BPW
    pltpu.sync_copy(ihd_hbm.at[pl.ds(base, BPW)], ihd_v)
    pltpu.sync_copy(iwm_hbm.at[pl.ds(base, BPW)], iwm_v)

    abufs = (a0, a1)
    bbufs = (b0, b1)
    obufs = (o0, o1)

    def start_gathers(t, phase):
        a, b = abufs[phase], bbufs[phase]
        pltpu.async_copy(ptf_hbm.at[ihd_v.at[pl.ds(t * G, G)]], a, sem_g)
        pltpu.async_copy(ptf_hbm.at[iwm_v.at[pl.ds(t * G, G)]], b, sem_g)

    def wait_one(sem, dst):
        pltpu.make_async_copy(ptf_hbm.at[pl.ds(0, G)], dst, sem).wait()

    def wait_out(phase):
        pltpu.make_async_copy(
            obufs[phase], out_hbm.at[pl.ds(0, G)], sem_o).wait()

    start_gathers(0, 0)

    def pair_body(k, _):
        for phase in range(2):
            t = 2 * k + phase
            a, b, o = abufs[phase], bbufs[phase], obufs[phase]
            wait_one(sem_g, a)
            wait_one(sem_g, b)

            @pl.when(t + 1 < CHUNKS)
            def _():
                start_gathers(t + 1, (phase + 1) % 2)

            @pl.when(t >= 2)
            def _():
                wait_out(phase)

            @plsc.parallel_loop(0, G)
            def row_body(r):
                for c in range(HIDDEN // 32):
                    va = plsc.bitcast(a[r, pl.ds(c * L, L)], jnp.bfloat16)
                    vb = plsc.bitcast(b[r, pl.ds(c * L, L)], jnp.bfloat16)
                    la, ha = plsc.unpack(
                        va, format=plsc.PackFormat.INTERLEAVED)
                    lb, hb = plsc.unpack(
                        vb, format=plsc.PackFormat.INTERLEAVED)
                    o[r, pl.ds(c * 32, L)] = la + lb
                    o[r, pl.ds(c * 32 + L, L)] = ha + hb

            pltpu.async_copy(o, out_hbm.at[pl.ds(base + t * G, G)], sem_o)
        return 0

    lax.fori_loop(0, CHUNKS // 2, pair_body, 0)
    wait_out(0)
    wait_out(1)


@jax.jit
def kernel(hours, days, weeks, months, hour_table, day_table, week_table,
           month_table, proj_w, proj_b):
    f32 = jnp.float32
    t = jnp.zeros((ROWS, HIDDEN), f32)
    t = t.at[0:24, 0:QUARTER].set(hour_table)
    t = t.at[24:31, QUARTER:2 * QUARTER].set(day_table)
    t = t.at[31:83, 2 * QUARTER:3 * QUARTER].set(week_table)
    t = t.at[83:95, 3 * QUARTER:4 * QUARTER].set(month_table)

    ptable = pl.pallas_call(
        _proj_body,
        out_shape=jax.ShapeDtypeStruct((NFULL, HIDDEN), jnp.bfloat16),
    )(t, proj_w, jnp.asarray(_E), proj_b.reshape(1, HIDDEN))
    ptable = (ptable.reshape(NFULL, HIDDEN // 32, 2, L)
              .transpose(0, 1, 3, 2).reshape(NFULL, HIDDEN // 2, 2))
    ptable = lax.bitcast_convert_type(ptable, f32)

    i32 = jnp.int32
    ihd = hours.astype(i32) * 7 + days.astype(i32)
    iwm = NHD + weeks.astype(i32) * 12 + months.astype(i32)

    mesh = plsc.VectorSubcoreMesh(core_axis_name="c", subcore_axis_name="s")
    sc = functools.partial(
        pl.kernel,
        out_type=jax.ShapeDtypeStruct((BATCH, HIDDEN), f32),
        mesh=mesh,
        compiler_params=pltpu.CompilerParams(needs_layout_passes=False),
        scratch_types=[
            pltpu.VMEM((BPW,), i32),
            pltpu.VMEM((BPW,), i32),
            pltpu.VMEM((G, HIDDEN // 2), f32),
            pltpu.VMEM((G, HIDDEN // 2), f32),
            pltpu.VMEM((G, HIDDEN // 2), f32),
            pltpu.VMEM((G, HIDDEN // 2), f32),
            pltpu.VMEM((G, HIDDEN), f32),
            pltpu.VMEM((G, HIDDEN), f32),
            pltpu.SemaphoreType.DMA,
            pltpu.SemaphoreType.DMA,
        ],
    )(_sc_body)
    return sc(ptable, ihd, iwm)
